# baseline (speedup 1.0000x reference)
import jax
import jax.numpy as jnp
from jax.experimental import pallas as pl
from jax.experimental.pallas import tpu as pltpu

_TOTAL_TOKENS = 32768
_BATCH = 16
_D_MODEL = 4096
_WAVE = _BATCH // 4


def _pooler_body(lens_ref, hs_ref, out_hbm, buf, in_sem, out_sem):
    running = lens_ref[0]
    for i in range(_BATCH):
        pltpu.make_async_copy(
            hs_ref.at[pl.ds(running - 1, 1)], buf.at[pl.ds(i, 1)], in_sem
        ).start()
        if i + 1 < _BATCH:
            running = running + lens_ref[i + 1]
    pltpu.make_async_copy(hs_ref.at[pl.ds(0, _BATCH)], buf, in_sem).wait()

    for h in range(4):
        rows = pl.ds(h * _WAVE, _WAVE)
        x = buf[rows, :]
        ss = jnp.sum(x * x, axis=1, keepdims=True)
        scale = jnp.where(ss > 1e-24, jax.lax.rsqrt(ss), 1e12)
        buf[rows, :] = x * scale
        pltpu.make_async_copy(buf.at[rows], out_hbm.at[rows], out_sem).start()
    pltpu.make_async_copy(buf, out_hbm, out_sem).wait()


def kernel(hidden_states, extend_seq_lens):
    return pl.pallas_call(
        _pooler_body,
        out_shape=jax.ShapeDtypeStruct((_BATCH, _D_MODEL), jnp.float32),
        in_specs=[
            pl.BlockSpec(memory_space=pltpu.SMEM),
            pl.BlockSpec(memory_space=pltpu.HBM),
        ],
        out_specs=pl.BlockSpec(memory_space=pltpu.HBM),
        scratch_shapes=[
            pltpu.VMEM((_BATCH, _D_MODEL), jnp.float32),
            pltpu.SemaphoreType.DMA,
            pltpu.SemaphoreType.DMA,
        ],
    )(extend_seq_lens, hidden_states)


# speedup vs baseline: 1.0172x; 1.0172x over previous
import jax
import jax.numpy as jnp
from jax.experimental import pallas as pl
from jax.experimental.pallas import tpu as pltpu

_TOTAL_TOKENS = 32768
_BATCH = 16
_D_MODEL = 4096
_HALF = _BATCH // 2


def _pooler_body(lens_ref, hs_ref, out_hbm, buf, in_sem, out_sem):
    running = lens_ref[0]
    for i in range(_BATCH):
        pltpu.make_async_copy(
            hs_ref.at[pl.ds(running - 1, 1)], buf.at[pl.ds(i, 1)], in_sem
        ).start()
        if i + 1 < _BATCH:
            running = running + lens_ref[i + 1]
    pltpu.make_async_copy(hs_ref.at[pl.ds(0, _BATCH)], buf, in_sem).wait()

    for h in range(2):
        rows = pl.ds(h * _HALF, _HALF)
        x = buf[rows, :]
        ss = jnp.sum(x * x, axis=1, keepdims=True)
        scale = jnp.where(ss > 1e-24, jax.lax.rsqrt(ss), 1e12)
        buf[rows, :] = x * scale
        pltpu.make_async_copy(buf.at[rows], out_hbm.at[rows], out_sem).start()
    pltpu.make_async_copy(buf, out_hbm, out_sem).wait()


def kernel(hidden_states, extend_seq_lens):
    return pl.pallas_call(
        _pooler_body,
        out_shape=jax.ShapeDtypeStruct((_BATCH, _D_MODEL), jnp.float32),
        in_specs=[
            pl.BlockSpec(memory_space=pltpu.SMEM),
            pl.BlockSpec(memory_space=pltpu.HBM),
        ],
        out_specs=pl.BlockSpec(memory_space=pltpu.HBM),
        scratch_shapes=[
            pltpu.VMEM((_BATCH, _D_MODEL), jnp.float32),
            pltpu.SemaphoreType.DMA,
            pltpu.SemaphoreType.DMA,
        ],
    )(extend_seq_lens, hidden_states)


# speedup vs baseline: 1.0281x; 1.0107x over previous
import jax
import jax.numpy as jnp
from jax.experimental import pallas as pl
from jax.experimental.pallas import tpu as pltpu

_TOTAL_TOKENS = 32768
_BATCH = 16
_D_MODEL = 4096
_HALF = _BATCH // 2


def _pooler_body(lens_ref, hs_ref, out_hbm, buf, in_sem, out_sem):
    running = lens_ref[0]
    for i in range(_BATCH):
        pltpu.make_async_copy(
            hs_ref.at[pl.ds(running - 1, 1)], buf.at[pl.ds(i, 1)], in_sem
        ).start()
        if i + 1 < _BATCH:
            running = running + lens_ref[i + 1]
    pltpu.make_async_copy(hs_ref.at[pl.ds(0, _BATCH)], buf, in_sem).wait()

    for h in range(2):
        rows = pl.ds(h * _HALF, _HALF)
        x = buf[rows, :]
        ss = jnp.sum(x * x, axis=1, keepdims=True)
        scale = jnp.where(ss > 1e-24, jax.lax.rsqrt(ss), 1e12)
        buf[rows, :] = x * scale
        pltpu.make_async_copy(buf.at[rows], out_hbm.at[rows], out_sem).start()
    pltpu.make_async_copy(buf, out_hbm, out_sem).wait()


def kernel(hidden_states, extend_seq_lens):
    grid_spec = pltpu.PrefetchScalarGridSpec(
        num_scalar_prefetch=1,
        grid=(1,),
        in_specs=[pl.BlockSpec(memory_space=pltpu.HBM)],
        out_specs=pl.BlockSpec(memory_space=pltpu.HBM),
        scratch_shapes=[
            pltpu.VMEM((_BATCH, _D_MODEL), jnp.float32),
            pltpu.SemaphoreType.DMA,
            pltpu.SemaphoreType.DMA,
        ],
    )
    return pl.pallas_call(
        _pooler_body,
        out_shape=jax.ShapeDtypeStruct((_BATCH, _D_MODEL), jnp.float32),
        grid_spec=grid_spec,
    )(extend_seq_lens, hidden_states)
